# Initial kernel scaffold; baseline (speedup 1.0000x reference)
import jax
import jax.numpy as jnp
from jax.experimental import pallas as pl


def kernel(pred, gt):
    raise NotImplementedError("write your pallas kernel here")


# speedup vs baseline: 55.1282x; 55.1282x over previous
import functools

import jax
import jax.numpy as jnp
from jax.experimental import pallas as pl
from jax.experimental.pallas import tpu as pltpu

_PNUM = 1024
_RB = 8


def _poly_loss_kernel(p_ref, g_ref, o_ref):
    px = p_ref[0, 0:1, :]
    py = p_ref[0, 1:2, :]
    gx = g_ref[0, 0:1, :]
    gy = g_ref[0, 1:2, :]

    def _roll(v, r):
        return v if r == 0 else jnp.roll(v, -r, axis=1)

    gx8 = jnp.concatenate([_roll(gx, r) for r in range(_RB)], axis=0)
    gy8 = jnp.concatenate([_roll(gy, r) for r in range(_RB)], axis=0)

    pxb = jnp.broadcast_to(px, (_RB, _PNUM))
    pyb = jnp.broadcast_to(py, (_RB, _PNUM))

    def smooth2(d):
        a = jnp.abs(d)
        m = jnp.minimum(a, 1.0)
        return m * (a + a - m)

    def body(_, carry):
        gxc, gyc, acc = carry
        f = smooth2(pxb - gxc) + smooth2(pyb - gyc)
        s = jnp.sum(f, axis=1, keepdims=True)
        acc = jnp.minimum(acc, s)
        gxc = jnp.roll(gxc, -_RB, axis=1)
        gyc = jnp.roll(gyc, -_RB, axis=1)
        return gxc, gyc, acc

    acc0 = jnp.full((_RB, 1), jnp.inf, dtype=jnp.float32)
    _, _, acc = jax.lax.fori_loop(
        0, _PNUM // _RB, body, (gx8, gy8, acc0)
    )
    o_ref[0, :, :] = jnp.min(acc, axis=(0, 1), keepdims=True)


@jax.jit
def kernel(pred, gt):
    b = pred.shape[0]
    p = jnp.transpose(pred, (0, 2, 1))
    g = jnp.transpose(gt, (0, 2, 1))
    mins = pl.pallas_call(
        _poly_loss_kernel,
        grid=(b,),
        in_specs=[
            pl.BlockSpec((1, 2, _PNUM), lambda i: (i, 0, 0)),
            pl.BlockSpec((1, 2, _PNUM), lambda i: (i, 0, 0)),
        ],
        out_specs=pl.BlockSpec((1, 1, 1), lambda i: (i, 0, 0)),
        out_shape=jax.ShapeDtypeStruct((b, 1, 1), jnp.float32),
        compiler_params=pltpu.CompilerParams(
            dimension_semantics=("parallel",),
        ),
    )(p, g)
    return jnp.mean(mins) / (2.0 * _PNUM)


# speedup vs baseline: 195.1738x; 3.5404x over previous
import functools

import jax
import jax.numpy as jnp
from jax.experimental import pallas as pl
from jax.experimental.pallas import tpu as pltpu

_PNUM = 1024
_RB = 8


def _poly_loss_kernel(p_ref, g_ref, o_ref):
    px = p_ref[0, 0:1, :]
    py = p_ref[0, 1:2, :]
    gx = g_ref[0, 0:1, :]
    gy = g_ref[0, 1:2, :]

    def _roll(v, r):
        return v if r == 0 else jnp.roll(v, -r, axis=1)

    gx8 = jnp.concatenate([_roll(gx, r) for r in range(_RB)], axis=0)
    gy8 = jnp.concatenate([_roll(gy, r) for r in range(_RB)], axis=0)

    pxb = jnp.broadcast_to(px, (_RB, _PNUM))
    pyb = jnp.broadcast_to(py, (_RB, _PNUM))

    def smooth2(d):
        a = jnp.abs(d)
        m = jnp.minimum(a, 1.0)
        return m * (a + a - m)

    def body(_, carry):
        gxc, gyc, acc = carry
        for o in range(_PNUM // 128):
            sh = 128 * o
            gxo = gxc if o == 0 else jnp.roll(gxc, -sh, axis=1)
            gyo = gyc if o == 0 else jnp.roll(gyc, -sh, axis=1)
            f = smooth2(pxb - gxo) + smooth2(pyb - gyo)
            s = jnp.sum(f, axis=1, keepdims=True)
            acc = jnp.minimum(acc, s)
        gxc = jnp.roll(gxc, -_RB, axis=1)
        gyc = jnp.roll(gyc, -_RB, axis=1)
        return gxc, gyc, acc

    acc0 = jnp.full((_RB, 1), jnp.inf, dtype=jnp.float32)
    _, _, acc = jax.lax.fori_loop(
        0, 128 // _RB, body, (gx8, gy8, acc0)
    )
    o_ref[0, :, :] = jnp.min(acc, axis=(0, 1), keepdims=True)


@jax.jit
def kernel(pred, gt):
    b = pred.shape[0]
    p = jnp.transpose(pred, (0, 2, 1))
    g = jnp.transpose(gt, (0, 2, 1))
    mins = pl.pallas_call(
        _poly_loss_kernel,
        grid=(b,),
        in_specs=[
            pl.BlockSpec((1, 2, _PNUM), lambda i: (i, 0, 0)),
            pl.BlockSpec((1, 2, _PNUM), lambda i: (i, 0, 0)),
        ],
        out_specs=pl.BlockSpec((1, 1, 1), lambda i: (i, 0, 0)),
        out_shape=jax.ShapeDtypeStruct((b, 1, 1), jnp.float32),
        compiler_params=pltpu.CompilerParams(
            dimension_semantics=("parallel",),
        ),
    )(p, g)
    return jnp.mean(mins) / (2.0 * _PNUM)


# speedup vs baseline: 195.2985x; 1.0006x over previous
import functools

import jax
import jax.numpy as jnp
from jax.experimental import pallas as pl
from jax.experimental.pallas import tpu as pltpu

_PNUM = 1024
_RB = 8


def _poly_loss_kernel(p_ref, g_ref, o_ref):
    px = p_ref[0, 0:1, :]
    py = p_ref[0, 1:2, :]
    gx = g_ref[0, 0:1, :]
    gy = g_ref[0, 1:2, :]

    def _roll(v, r):
        return v if r == 0 else jnp.roll(v, -r, axis=1)

    gx8 = jnp.concatenate([_roll(gx, r) for r in range(_RB)], axis=0)
    gy8 = jnp.concatenate([_roll(gy, r) for r in range(_RB)], axis=0)

    pxb = jnp.broadcast_to(px, (_RB, _PNUM))
    pyb = jnp.broadcast_to(py, (_RB, _PNUM))
    pxo = [pxb] + [jnp.roll(pxb, 128 * o, axis=1) for o in range(1, _PNUM // 128)]
    pyo = [pyb] + [jnp.roll(pyb, 128 * o, axis=1) for o in range(1, _PNUM // 128)]

    def smooth2(d):
        a = jnp.abs(d)
        m = jnp.minimum(a, 1.0)
        return m * (a + a - m)

    def body(_, carry):
        gxc, gyc, acc = carry
        for o in range(_PNUM // 128):
            f = smooth2(pxo[o] - gxc) + smooth2(pyo[o] - gyc)
            s = jnp.sum(f, axis=1, keepdims=True)
            acc = jnp.minimum(acc, s)
        gxc = jnp.roll(gxc, -_RB, axis=1)
        gyc = jnp.roll(gyc, -_RB, axis=1)
        return gxc, gyc, acc

    acc0 = jnp.full((_RB, 1), jnp.inf, dtype=jnp.float32)
    _, _, acc = jax.lax.fori_loop(
        0, 128 // _RB, body, (gx8, gy8, acc0)
    )
    o_ref[0, :, :] = jnp.min(acc, axis=(0, 1), keepdims=True)


@jax.jit
def kernel(pred, gt):
    b = pred.shape[0]
    p = jnp.transpose(pred, (0, 2, 1))
    g = jnp.transpose(gt, (0, 2, 1))
    mins = pl.pallas_call(
        _poly_loss_kernel,
        grid=(b,),
        in_specs=[
            pl.BlockSpec((1, 2, _PNUM), lambda i: (i, 0, 0)),
            pl.BlockSpec((1, 2, _PNUM), lambda i: (i, 0, 0)),
        ],
        out_specs=pl.BlockSpec((1, 1, 1), lambda i: (i, 0, 0)),
        out_shape=jax.ShapeDtypeStruct((b, 1, 1), jnp.float32),
        compiler_params=pltpu.CompilerParams(
            dimension_semantics=("parallel",),
        ),
    )(p, g)
    return jnp.mean(mins) / (2.0 * _PNUM)
